# baseline (speedup 1.0000x reference)
import functools

import jax
import jax.numpy as jnp
from jax import lax
from jax.experimental import pallas as pl
from jax.experimental.pallas import tpu as pltpu
from jax.experimental.pallas import tpu_sc as plsc


def kernel(feature, row_embed, col_embed):
    B, C, H, W = feature.shape
    half = C // 2
    info = plsc.get_sparse_core_info()
    n_workers = info.num_cores * info.num_subcores
    h_per_w = H // n_workers
    mesh = plsc.VectorSubcoreMesh(core_axis_name="c", subcore_axis_name="s")

    @functools.partial(
        pl.kernel,
        mesh=mesh,
        out_type=jax.ShapeDtypeStruct((B, H, W, C), row_embed.dtype),
        scratch_types=[
            pltpu.VMEM((W, half), jnp.float32),
            pltpu.VMEM((h_per_w, W, half), jnp.float32),
            pltpu.VMEM((W,), jnp.int32),
            pltpu.SemaphoreType.DMA,
        ],
        compiler_params=pltpu.CompilerParams(use_tc_tiling_on_sc=False),
    )
    def _sc_pos_broadcast(row_hbm, col_hbm, out_hbm, colbuf, rowbuf, idx_v, sem):
        wid = lax.axis_index("s") * info.num_cores + lax.axis_index("c")
        pltpu.sync_copy(col_hbm.at[pl.ds(0, W), :], colbuf)
        for g in range(h_per_w):
            h = wid * h_per_w + g
            for t in range(W // 16):
                idx_v[pl.ds(16 * t, 16)] = jnp.full((16,), h, jnp.int32)
            pltpu.async_copy(row_hbm.at[idx_v], rowbuf.at[g], sem).wait()
        pending = []
        for g in range(h_per_w):
            h = wid * h_per_w + g
            for b in range(B):
                pending.append(
                    pltpu.async_copy(
                        colbuf, out_hbm.at[b, h, :, pl.ds(0, half)], sem
                    )
                )
                pending.append(
                    pltpu.async_copy(
                        rowbuf.at[g], out_hbm.at[b, h, :, pl.ds(half, half)], sem
                    )
                )
        for d in pending:
            d.wait()

    out = _sc_pos_broadcast(row_embed, col_embed)
    return jnp.transpose(out, (0, 3, 1, 2))


# speedup vs baseline: 6.7860x; 6.7860x over previous
import jax
import jax.numpy as jnp
from jax.experimental import pallas as pl
from jax.experimental.pallas import tpu as pltpu


def _make_pos_broadcast_kernel(B, H, W, half):
    n_chunks = 4
    hh = H // n_chunks

    def _pos_broadcast_kernel(row_t_ref, col_t_ref, out_ref, scratch, sem):
        cols = col_t_ref[:, :W].T
        rows = row_t_ref[:, :H].T
        for k in range(n_chunks):
            sl = pl.ds(k * hh, hh)
            scratch[sl, :, :half] = jnp.broadcast_to(cols[None], (hh, W, half))
            scratch[sl, :, half:] = jnp.broadcast_to(
                rows[k * hh : (k + 1) * hh, None, :], (hh, W, half)
            )
            for b in range(B):
                pltpu.make_async_copy(
                    scratch.at[sl], out_ref.at[b, sl], sem
                ).start()
        for _ in range(B * n_chunks):
            pltpu.make_async_copy(
                scratch.at[pl.ds(0, hh)], out_ref.at[0, pl.ds(0, hh)], sem
            ).wait()

    return _pos_broadcast_kernel


def kernel(feature, row_embed, col_embed):
    B, C, H, W = feature.shape
    half = C // 2
    out = pl.pallas_call(
        _make_pos_broadcast_kernel(B, H, W, half),
        in_specs=[
            pl.BlockSpec(memory_space=pltpu.MemorySpace.VMEM),
            pl.BlockSpec(memory_space=pltpu.MemorySpace.VMEM),
        ],
        out_specs=pl.BlockSpec(memory_space=pl.ANY),
        out_shape=jax.ShapeDtypeStruct((B, H, W, C), row_embed.dtype),
        scratch_shapes=[
            pltpu.VMEM((H, W, C), row_embed.dtype),
            pltpu.SemaphoreType.DMA,
        ],
    )(row_embed.T, col_embed.T)
    return jnp.transpose(out, (0, 3, 1, 2))
